# Initial kernel scaffold; baseline (speedup 1.0000x reference)
import jax
import jax.numpy as jnp
from jax.experimental import pallas as pl


def kernel(boxes, scores):
    raise NotImplementedError("write your pallas kernel here")


# speedup vs baseline: 361.7692x; 361.7692x over previous
import jax
import jax.numpy as jnp
from jax import lax
from jax.experimental import pallas as pl
from jax.experimental.pallas import tpu as pltpu

_B = 512
_SCORE_T = 0.5
_IOU_T = 0.7


def _iou_tile(rx1, ry1, rx2, ry2, ra, cx1, cy1, cx2, cy2, ca):
    ix1 = jnp.maximum(rx1, cx1)
    iy1 = jnp.maximum(ry1, cy1)
    ix2 = jnp.minimum(rx2, cx2)
    iy2 = jnp.minimum(ry2, cy2)
    inter = jnp.maximum(ix2 - ix1, 0.0) * jnp.maximum(iy2 - iy1, 0.0)
    return inter / (ra + ca - inter + 1e-9)


def _matvec(k, mat):
    return lax.dot_general(k, mat, (((1,), (0,)), ((), ())),
                           preferred_element_type=jnp.float32)


def _nms_body(nb_ref, rows_ref, cols_ref, keep_ref):
    keep_ref[...] = rows_ref[4:5, :]
    nb = nb_ref[0]

    row_i = lax.broadcasted_iota(jnp.int32, (_B, _B), 0)
    col_i = lax.broadcasted_iota(jnp.int32, (_B, _B), 1)
    upper = row_i < col_i

    def outer(j, _):
        s = j * _B
        jx1 = cols_ref[pl.ds(s, _B), 0:1]
        jy1 = cols_ref[pl.ds(s, _B), 1:2]
        jx2 = cols_ref[pl.ds(s, _B), 2:3]
        jy2 = cols_ref[pl.ds(s, _B), 3:4]
        ja = (jx2 - jx1) * (jy2 - jy1)
        rx1 = rows_ref[0:1, pl.ds(s, _B)]
        ry1 = rows_ref[1:2, pl.ds(s, _B)]
        rx2 = rows_ref[2:3, pl.ds(s, _B)]
        ry2 = rows_ref[3:4, pl.ds(s, _B)]
        rar = (rx2 - rx1) * (ry2 - ry1)

        iou_jj = _iou_tile(jx1, jy1, jx2, jy2, ja, rx1, ry1, rx2, ry2, rar)
        confl = jnp.where((iou_jj > _IOU_T) & upper, 1.0, 0.0)

        init = keep_ref[0:1, pl.ds(s, _B)]

        def w_cond(carry):
            return carry[1]

        def w_body(carry):
            k, _ = carry
            cnt = _matvec(k, confl)
            nk = jnp.where(cnt > 0.5, 0.0, init)
            changed = jnp.sum(jnp.abs(nk - k)) > 0.0
            return (nk, changed)

        kj, _ = lax.while_loop(w_cond, w_body, (init, jnp.bool_(True)))
        keep_ref[0:1, pl.ds(s, _B)] = kj

        def inner(t, _):
            u = t * _B
            cx1 = rows_ref[0:1, pl.ds(u, _B)]
            cy1 = rows_ref[1:2, pl.ds(u, _B)]
            cx2 = rows_ref[2:3, pl.ds(u, _B)]
            cy2 = rows_ref[3:4, pl.ds(u, _B)]
            car = (cx2 - cx1) * (cy2 - cy1)
            iou = _iou_tile(jx1, jy1, jx2, jy2, ja, cx1, cy1, cx2, cy2, car)
            confl2 = jnp.where(iou > _IOU_T, 1.0, 0.0)
            cnt = _matvec(kj, confl2)
            cur = keep_ref[0:1, pl.ds(u, _B)]
            keep_ref[0:1, pl.ds(u, _B)] = jnp.where(cnt > 0.5, 0.0, cur)
            return 0

        lax.fori_loop(j + 1, nb, inner, 0)
        return 0

    lax.fori_loop(0, nb, outer, 0)


def _run_nms(nb_arr, rows, cols):
    npad = rows.shape[1]
    return pl.pallas_call(
        _nms_body,
        out_shape=jax.ShapeDtypeStruct((1, npad), jnp.float32),
        in_specs=[
            pl.BlockSpec(memory_space=pltpu.SMEM),
            pl.BlockSpec(memory_space=pltpu.VMEM),
            pl.BlockSpec(memory_space=pltpu.VMEM),
        ],
        out_specs=pl.BlockSpec(memory_space=pltpu.VMEM),
    )(nb_arr, rows, cols)


def kernel(boxes, scores):
    n = boxes.shape[0]
    npad = ((n + _B - 1) // _B) * _B
    valid = scores > _SCORE_T
    key = jnp.where(valid, -scores, jnp.inf)
    order = jnp.argsort(key)
    boxes_s = jnp.take(boxes, order, axis=0)
    valid_s = jnp.take(valid, order).astype(jnp.float32)
    nv = jnp.sum(valid.astype(jnp.int32))
    nb_arr = ((nv + _B - 1) // _B).reshape((1,))

    rows = jnp.zeros((8, npad), jnp.float32)
    rows = rows.at[0:4, :n].set(boxes_s.T)
    rows = rows.at[4, :n].set(valid_s)
    cols = rows.T

    keep_s = _run_nms(nb_arr, rows, cols)[0, :n]
    keep = jnp.zeros((n,), jnp.float32).at[order].set(keep_s)
    return jnp.concatenate([boxes * keep[:, None], (scores * keep)[:, None]],
                           axis=1)


# speedup vs baseline: 397.9897x; 1.1001x over previous
import jax
import jax.numpy as jnp
from jax import lax
from jax.experimental import pallas as pl
from jax.experimental.pallas import tpu as pltpu

_B = 512
_SCORE_T = 0.5
_IOU_T = 0.7


def _iou_tile(rx1, ry1, rx2, ry2, ra, cx1, cy1, cx2, cy2, ca):
    ix1 = jnp.maximum(rx1, cx1)
    iy1 = jnp.maximum(ry1, cy1)
    ix2 = jnp.minimum(rx2, cx2)
    iy2 = jnp.minimum(ry2, cy2)
    inter = jnp.maximum(ix2 - ix1, 0.0) * jnp.maximum(iy2 - iy1, 0.0)
    return inter / (ra + ca - inter + 1e-9)


def _matvec(k, mat):
    return lax.dot_general(k, mat, (((1,), (0,)), ((), ())),
                           preferred_element_type=jnp.float32)


def _nms_body(scal_ref, rows_ref, cols_ref, keep_ref):
    nb = scal_ref[0]
    nv = scal_ref[1]
    npad = keep_ref.shape[1]
    pos = lax.broadcasted_iota(jnp.int32, (1, npad), 1)
    keep_ref[...] = jnp.where(pos < nv, 1.0, 0.0)

    row_i = lax.broadcasted_iota(jnp.int32, (_B, _B), 0)
    col_i = lax.broadcasted_iota(jnp.int32, (_B, _B), 1)
    upper = row_i < col_i

    def outer(j, _):
        s = j * _B
        jx1 = cols_ref[pl.ds(s, _B), 0:1]
        jy1 = cols_ref[pl.ds(s, _B), 1:2]
        jx2 = cols_ref[pl.ds(s, _B), 2:3]
        jy2 = cols_ref[pl.ds(s, _B), 3:4]
        ja = (jx2 - jx1) * (jy2 - jy1)
        rx1 = rows_ref[0:1, pl.ds(s, _B)]
        ry1 = rows_ref[1:2, pl.ds(s, _B)]
        rx2 = rows_ref[2:3, pl.ds(s, _B)]
        ry2 = rows_ref[3:4, pl.ds(s, _B)]
        rar = (rx2 - rx1) * (ry2 - ry1)

        iou_jj = _iou_tile(jx1, jy1, jx2, jy2, ja, rx1, ry1, rx2, ry2, rar)
        confl = jnp.where((iou_jj > _IOU_T) & upper, 1.0, 0.0)

        init = keep_ref[0:1, pl.ds(s, _B)]

        def w_cond(carry):
            return carry[1]

        def w_body(carry):
            k, _ = carry
            cnt = _matvec(k, confl)
            nk = jnp.where(cnt > 0.5, 0.0, init)
            changed = jnp.sum(jnp.abs(nk - k)) > 0.0
            return (nk, changed)

        kj, _ = lax.while_loop(w_cond, w_body, (init, jnp.bool_(True)))
        keep_ref[0:1, pl.ds(s, _B)] = kj

        def inner(t, _):
            u = t * _B
            cx1 = rows_ref[0:1, pl.ds(u, _B)]
            cy1 = rows_ref[1:2, pl.ds(u, _B)]
            cx2 = rows_ref[2:3, pl.ds(u, _B)]
            cy2 = rows_ref[3:4, pl.ds(u, _B)]
            car = (cx2 - cx1) * (cy2 - cy1)
            iou = _iou_tile(jx1, jy1, jx2, jy2, ja, cx1, cy1, cx2, cy2, car)
            confl2 = jnp.where(iou > _IOU_T, 1.0, 0.0)
            cnt = _matvec(kj, confl2)
            cur = keep_ref[0:1, pl.ds(u, _B)]
            keep_ref[0:1, pl.ds(u, _B)] = jnp.where(cnt > 0.5, 0.0, cur)
            return 0

        lax.fori_loop(j + 1, nb, inner, 0)
        return 0

    lax.fori_loop(0, nb, outer, 0)


def _run_nms(scal, rows, cols):
    npad = rows.shape[1]
    return pl.pallas_call(
        _nms_body,
        out_shape=jax.ShapeDtypeStruct((1, npad), jnp.float32),
        in_specs=[
            pl.BlockSpec(memory_space=pltpu.SMEM),
            pl.BlockSpec(memory_space=pltpu.VMEM),
            pl.BlockSpec(memory_space=pltpu.VMEM),
        ],
        out_specs=pl.BlockSpec(memory_space=pltpu.VMEM),
    )(scal, rows, cols)


def kernel(boxes, scores):
    n = boxes.shape[0]
    npad = ((n + _B - 1) // _B) * _B
    valid = scores > _SCORE_T
    key = jnp.where(valid, -scores, jnp.inf)
    order = jnp.argsort(key)
    boxes_s = jnp.take(boxes, order, axis=0)
    nv = jnp.sum(valid.astype(jnp.int32))
    nb = (nv + _B - 1) // _B
    scal = jnp.stack([nb, nv])

    cols = jnp.zeros((npad, 8), jnp.float32).at[:n, 0:4].set(boxes_s)
    rows = jnp.zeros((4, npad), jnp.float32).at[:, :n].set(boxes_s.T)

    keep_s = _run_nms(scal, rows, cols)[0, :n]
    keep = jnp.zeros((n,), jnp.float32).at[order].set(keep_s)
    return jnp.concatenate([boxes * keep[:, None], (scores * keep)[:, None]],
                           axis=1)


# speedup vs baseline: 430.3198x; 1.0812x over previous
import jax
import jax.numpy as jnp
from jax import lax
from jax.experimental import pallas as pl
from jax.experimental.pallas import tpu as pltpu

_B = 1024
_SCORE_T = 0.5
_IOU_T = 0.7


def _iou_tile(rx1, ry1, rx2, ry2, ra, cx1, cy1, cx2, cy2, ca):
    ix1 = jnp.maximum(rx1, cx1)
    iy1 = jnp.maximum(ry1, cy1)
    ix2 = jnp.minimum(rx2, cx2)
    iy2 = jnp.minimum(ry2, cy2)
    inter = jnp.maximum(ix2 - ix1, 0.0) * jnp.maximum(iy2 - iy1, 0.0)
    return inter / (ra + ca - inter + 1e-9)


def _matvec(k, mat):
    return lax.dot_general(k, mat, (((1,), (0,)), ((), ())),
                           preferred_element_type=jnp.float32)


def _nms_body(scal_ref, rows_ref, cols_ref, keep_ref):
    nb = scal_ref[0]
    nv = scal_ref[1]
    npad = keep_ref.shape[1]
    pos = lax.broadcasted_iota(jnp.int32, (1, npad), 1)
    keep_ref[...] = jnp.where(pos < nv, 1.0, 0.0)

    row_i = lax.broadcasted_iota(jnp.int32, (_B, _B), 0)
    col_i = lax.broadcasted_iota(jnp.int32, (_B, _B), 1)
    upper = row_i < col_i

    def outer(j, _):
        s = j * _B
        jx1 = cols_ref[pl.ds(s, _B), 0:1]
        jy1 = cols_ref[pl.ds(s, _B), 1:2]
        jx2 = cols_ref[pl.ds(s, _B), 2:3]
        jy2 = cols_ref[pl.ds(s, _B), 3:4]
        ja = (jx2 - jx1) * (jy2 - jy1)
        rx1 = rows_ref[0:1, pl.ds(s, _B)]
        ry1 = rows_ref[1:2, pl.ds(s, _B)]
        rx2 = rows_ref[2:3, pl.ds(s, _B)]
        ry2 = rows_ref[3:4, pl.ds(s, _B)]
        rar = (rx2 - rx1) * (ry2 - ry1)

        iou_jj = _iou_tile(jx1, jy1, jx2, jy2, ja, rx1, ry1, rx2, ry2, rar)
        confl = jnp.where((iou_jj > _IOU_T) & upper, 1.0, 0.0)

        init = keep_ref[0:1, pl.ds(s, _B)]

        def w_cond(carry):
            return carry[1]

        def w_body(carry):
            k, _ = carry
            cnt = _matvec(k, confl)
            nk = jnp.where(cnt > 0.5, 0.0, init)
            changed = jnp.sum(jnp.abs(nk - k)) > 0.0
            return (nk, changed)

        kj, _ = lax.while_loop(w_cond, w_body, (init, jnp.bool_(True)))
        keep_ref[0:1, pl.ds(s, _B)] = kj

        def inner(t, _):
            u = t * _B
            cx1 = rows_ref[0:1, pl.ds(u, _B)]
            cy1 = rows_ref[1:2, pl.ds(u, _B)]
            cx2 = rows_ref[2:3, pl.ds(u, _B)]
            cy2 = rows_ref[3:4, pl.ds(u, _B)]
            car = (cx2 - cx1) * (cy2 - cy1)
            iou = _iou_tile(jx1, jy1, jx2, jy2, ja, cx1, cy1, cx2, cy2, car)
            confl2 = jnp.where(iou > _IOU_T, 1.0, 0.0)
            cnt = _matvec(kj, confl2)
            cur = keep_ref[0:1, pl.ds(u, _B)]
            keep_ref[0:1, pl.ds(u, _B)] = jnp.where(cnt > 0.5, 0.0, cur)
            return 0

        lax.fori_loop(j + 1, nb, inner, 0)
        return 0

    lax.fori_loop(0, nb, outer, 0)


def _run_nms(scal, rows, cols):
    npad = rows.shape[1]
    return pl.pallas_call(
        _nms_body,
        out_shape=jax.ShapeDtypeStruct((1, npad), jnp.float32),
        in_specs=[
            pl.BlockSpec(memory_space=pltpu.SMEM),
            pl.BlockSpec(memory_space=pltpu.VMEM),
            pl.BlockSpec(memory_space=pltpu.VMEM),
        ],
        out_specs=pl.BlockSpec(memory_space=pltpu.VMEM),
    )(scal, rows, cols)


def kernel(boxes, scores):
    n = boxes.shape[0]
    npad = ((n + _B - 1) // _B) * _B
    valid = scores > _SCORE_T
    key = jnp.where(valid, -scores, jnp.inf)
    order = jnp.argsort(key)
    boxes_s = jnp.take(boxes, order, axis=0)
    nv = jnp.sum(valid.astype(jnp.int32))
    nb = (nv + _B - 1) // _B
    scal = jnp.stack([nb, nv])

    cols = jnp.zeros((npad, 8), jnp.float32).at[:n, 0:4].set(boxes_s)
    rows = jnp.zeros((4, npad), jnp.float32).at[:, :n].set(boxes_s.T)

    keep_s = _run_nms(scal, rows, cols)[0, :n]
    keep = jnp.zeros((n,), jnp.float32).at[order].set(keep_s)
    return jnp.concatenate([boxes * keep[:, None], (scores * keep)[:, None]],
                           axis=1)
